# baseline (speedup 1.0000x reference)
import functools

import jax
import jax.numpy as jnp
from jax import lax
from jax.experimental import pallas as pl
from jax.experimental.pallas import tpu as pltpu
from jax.experimental.pallas import tpu_sc as plsc

_B, _F, _V, _D = 4096, 26, 100000, 32
_DENSE_IN = 13
_FP = 32
_VBLK = 5000
_BBLK = 2048
_NTILES = 32
_BCHUNK = _B // _NTILES


_VP = 100096
_TSBLK = 5888
_TSGRID = _VP // _TSBLK


def _project_body(wft_ref, embt_ref, out_ref):
    out_ref[...] = lax.dot_general(wft_ref[...], embt_ref[...],
                                   dimension_numbers=(((0,), (0,)), ((), ())),
                                   preferred_element_type=jnp.float32)


def _project_table(embt, wft):
    return pl.pallas_call(
        _project_body,
        grid=(_TSGRID,),
        in_specs=[
            pl.BlockSpec((_D, _FP), lambda i: (0, 0)),
            pl.BlockSpec((_D, _TSBLK), lambda i: (0, i)),
        ],
        out_specs=pl.BlockSpec((_FP, _TSBLK), lambda i: (0, i)),
        out_shape=jax.ShapeDtypeStruct((_FP, _VP), jnp.float32),
    )(wft, embt)


def _mlp_body(x_ref, w1_ref, b1_ref, w2_ref, b2_ref, w3_ref, b3_ref,
              woh_ref, bo_ref, out_ref):
    h = jnp.maximum(jnp.dot(x_ref[...], w1_ref[...],
                            preferred_element_type=jnp.float32) + b1_ref[...], 0.0)
    h = jnp.maximum(jnp.dot(h, w2_ref[...],
                            preferred_element_type=jnp.float32) + b2_ref[...], 0.0)
    h = jnp.maximum(jnp.dot(h, w3_ref[...],
                            preferred_element_type=jnp.float32) + b3_ref[...], 0.0)
    out_ref[...] = jnp.sum(h * woh_ref[...], axis=1) + bo_ref[0, 0]


def _dense_tower(x, W1, b1, W2, b2, W3, b3, woh, bo):
    full = lambda shape: pl.BlockSpec(shape, lambda i: (0,) * len(shape))
    return pl.pallas_call(
        _mlp_body,
        grid=(_B // _BBLK,),
        in_specs=[
            pl.BlockSpec((_BBLK, _DENSE_IN), lambda i: (i, 0)),
            full((_DENSE_IN, 512)), full((1, 512)),
            full((512, 256)), full((1, 256)),
            full((256, _D)), full((1, _D)),
            full((1, _D)), full((1, 1)),
        ],
        out_specs=pl.BlockSpec((_BBLK,), lambda i: (i,)),
        out_shape=jax.ShapeDtypeStruct((_B,), jnp.float32),
    )(x, W1, b1, W2, b2, W3, b3, woh, bo)


def _sc_gather_body(adj_hbm, ts_hbm, hsum_hbm, out_hbm, idx_v, g_v, h_v, o_v, sem):
    w = lax.axis_index("s") * 2 + lax.axis_index("c")
    b0 = w * _BCHUNK
    pltpu.sync_copy(adj_hbm.at[w], idx_v)
    pltpu.sync_copy(hsum_hbm.at[pl.ds(b0, _BCHUNK)], h_v)
    for half in range(2):
        cps = [
            pltpu.async_copy(ts_hbm.at[idx_v.at[half * 13 + f]],
                             g_v.at[half * 13 + f], sem)
            for f in range(13)
        ]
        for cp in cps:
            cp.wait()
    for i in range(_BCHUNK // 16):
        sl = pl.ds(i * 16, 16)
        acc = h_v[sl]
        for f in range(_F):
            acc = acc + g_v[f, sl]
        o_v[sl] = acc
    pltpu.sync_copy(o_v, out_hbm.at[pl.ds(b0, _BCHUNK)])


def _sc_gather():
    return pl.kernel(
        _sc_gather_body,
        out_type=jax.ShapeDtypeStruct((_B,), jnp.float32),
        mesh=plsc.VectorSubcoreMesh(core_axis_name="c", subcore_axis_name="s",
                                    num_cores=2, num_subcores=16),
        scratch_types=[
            pltpu.VMEM((_F, _BCHUNK), jnp.int32),
            pltpu.VMEM((_F, _BCHUNK), jnp.float32),
            pltpu.VMEM((_BCHUNK,), jnp.float32),
            pltpu.VMEM((_BCHUNK,), jnp.float32),
            pltpu.SemaphoreType.DMA,
        ],
    )


def kernel(dense_features, sparse_indices, emb_table, W1, b1, W2, b2, W3, b3,
           Wo, bo):
    wf = Wo[: _F * _D, 0].reshape(_F, _D)
    wft = jnp.zeros((_D, _FP), jnp.float32).at[:, :_F].set(wf.T)
    woh = Wo[_F * _D:, 0].reshape(1, _D)

    ts = _project_table(emb_table.T, wft)
    ts_flat = ts.T.reshape(_VP * _FP)
    hsum = _dense_tower(dense_features, W1, b1.reshape(1, 512), W2,
                        b2.reshape(1, 256), W3, b3.reshape(1, _D), woh,
                        bo.reshape(1, 1))

    adj = sparse_indices * _FP + jnp.arange(_F, dtype=jnp.int32)[None, :]
    adj3 = adj.reshape(_NTILES, _BCHUNK, _F).transpose(0, 2, 1)

    out = _sc_gather()(adj3, ts_flat, hsum)
    return out.reshape(_B, 1)


# speedup vs baseline: 1.2790x; 1.2790x over previous
import functools

import jax
import jax.numpy as jnp
from jax import lax
from jax.experimental import pallas as pl
from jax.experimental.pallas import tpu as pltpu
from jax.experimental.pallas import tpu_sc as plsc

_B, _F, _V, _D = 4096, 26, 100000, 32
_DENSE_IN = 13
_FP = 32
_VBLK = 5000
_BBLK = 2048
_NTILES = 32
_BCHUNK = _B // _NTILES


_VP = 100096
_TSBLK = 5888
_TSGRID = _VP // _TSBLK


def _project_body(wft_ref, embt_ref, out_ref, rv_ref):
    rv_ref[...] = lax.dot_general(embt_ref[...], wft_ref[...],
                                  dimension_numbers=(((0,), (0,)), ((), ())),
                                  preferred_element_type=jnp.float32)
    for j in range(4):
        out_ref[:, _FP * j:_FP * (j + 1)] = rv_ref[pl.Slice(j, _TSBLK // 4, 4), :]


def _project_table(embt, wft):
    return pl.pallas_call(
        _project_body,
        grid=(_TSGRID,),
        in_specs=[
            pl.BlockSpec((_D, _FP), lambda i: (0, 0)),
            pl.BlockSpec((_D, _TSBLK), lambda i: (0, i)),
        ],
        out_specs=pl.BlockSpec((_TSBLK // 4, 4 * _FP), lambda i: (i, 0)),
        out_shape=jax.ShapeDtypeStruct((_VP // 4, 4 * _FP), jnp.float32),
        scratch_shapes=[pltpu.VMEM((_TSBLK, _FP), jnp.float32)],
    )(wft, embt)


def _mlp_body(x_ref, w1_ref, b1_ref, w2_ref, b2_ref, w3_ref, b3_ref,
              woh_ref, bo_ref, out_ref):
    h = jnp.maximum(jnp.dot(x_ref[...], w1_ref[...],
                            preferred_element_type=jnp.float32) + b1_ref[...], 0.0)
    h = jnp.maximum(jnp.dot(h, w2_ref[...],
                            preferred_element_type=jnp.float32) + b2_ref[...], 0.0)
    h = jnp.maximum(jnp.dot(h, w3_ref[...],
                            preferred_element_type=jnp.float32) + b3_ref[...], 0.0)
    out_ref[...] = jnp.sum(h * woh_ref[...], axis=1) + bo_ref[0, 0]


def _dense_tower(x, W1, b1, W2, b2, W3, b3, woh, bo):
    full = lambda shape: pl.BlockSpec(shape, lambda i: (0,) * len(shape))
    return pl.pallas_call(
        _mlp_body,
        grid=(_B // _BBLK,),
        in_specs=[
            pl.BlockSpec((_BBLK, _DENSE_IN), lambda i: (i, 0)),
            full((_DENSE_IN, 512)), full((1, 512)),
            full((512, 256)), full((1, 256)),
            full((256, _D)), full((1, _D)),
            full((1, _D)), full((1, 1)),
        ],
        out_specs=pl.BlockSpec((_BBLK,), lambda i: (i,)),
        out_shape=jax.ShapeDtypeStruct((_B,), jnp.float32),
    )(x, W1, b1, W2, b2, W3, b3, woh, bo)


def _sc_gather_body(adj_hbm, ts_hbm, hsum_hbm, out_hbm, idx_v, g_v, h_v, o_v, sem):
    w = lax.axis_index("s") * 2 + lax.axis_index("c")
    b0 = w * _BCHUNK
    pltpu.sync_copy(adj_hbm.at[w], idx_v)
    pltpu.sync_copy(hsum_hbm.at[pl.ds(b0, _BCHUNK)], h_v)
    for half in range(2):
        cps = [
            pltpu.async_copy(ts_hbm.at[idx_v.at[half * 13 + f]],
                             g_v.at[half * 13 + f], sem)
            for f in range(13)
        ]
        for cp in cps:
            cp.wait()
    for i in range(_BCHUNK // 16):
        sl = pl.ds(i * 16, 16)
        acc = h_v[sl]
        for f in range(_F):
            acc = acc + g_v[f, sl]
        o_v[sl] = acc
    pltpu.sync_copy(o_v, out_hbm.at[pl.ds(b0, _BCHUNK)])


def _sc_gather():
    return pl.kernel(
        _sc_gather_body,
        out_type=jax.ShapeDtypeStruct((_B,), jnp.float32),
        mesh=plsc.VectorSubcoreMesh(core_axis_name="c", subcore_axis_name="s",
                                    num_cores=2, num_subcores=16),
        scratch_types=[
            pltpu.VMEM((_F, _BCHUNK), jnp.int32),
            pltpu.VMEM((_F, _BCHUNK), jnp.float32),
            pltpu.VMEM((_BCHUNK,), jnp.float32),
            pltpu.VMEM((_BCHUNK,), jnp.float32),
            pltpu.SemaphoreType.DMA,
        ],
    )


def kernel(dense_features, sparse_indices, emb_table, W1, b1, W2, b2, W3, b3,
           Wo, bo):
    wf = Wo[: _F * _D, 0].reshape(_F, _D)
    wft = jnp.zeros((_D, _FP), jnp.float32).at[:, :_F].set(wf.T)
    woh = Wo[_F * _D:, 0].reshape(1, _D)

    ts = _project_table(emb_table.T, wft)
    ts_flat = ts.reshape(_VP * _FP)
    hsum = _dense_tower(dense_features, W1, b1.reshape(1, 512), W2,
                        b2.reshape(1, 256), W3, b3.reshape(1, _D), woh,
                        bo.reshape(1, 1))

    adj = sparse_indices * _FP + jnp.arange(_F, dtype=jnp.int32)[None, :]
    adj3 = adj.reshape(_NTILES, _BCHUNK, _F).transpose(0, 2, 1)

    out = _sc_gather()(adj3, ts_flat, hsum)
    return out.reshape(_B, 1)


# speedup vs baseline: 1.5410x; 1.2048x over previous
import functools

import jax
import jax.numpy as jnp
from jax import lax
from jax.experimental import pallas as pl
from jax.experimental.pallas import tpu as pltpu
from jax.experimental.pallas import tpu_sc as plsc

_B, _F, _V, _D = 4096, 26, 100000, 32
_DENSE_IN = 13
_FP = 32
_VBLK = 5000
_BBLK = 2048
_NTILES = 32
_BCHUNK = _B // _NTILES


_VP = 100096
_TSBLK = 5888
_TSGRID = _VP // _TSBLK


def _project_body(wft_ref, embt_ref, out_ref, tsb_ref):
    tsb_ref[...] = lax.dot_general(wft_ref[...], embt_ref[...],
                                   dimension_numbers=(((0,), (0,)), ((), ())),
                                   preferred_element_type=jnp.float32)
    for c in range(_TSBLK // 128):
        out_ref[c, :, :] = tsb_ref[:, 128 * c:128 * (c + 1)]


def _project_table(embt, wft):
    return pl.pallas_call(
        _project_body,
        grid=(_TSGRID,),
        in_specs=[
            pl.BlockSpec((_D, _FP), lambda i: (0, 0)),
            pl.BlockSpec((_D, _TSBLK), lambda i: (0, i)),
        ],
        out_specs=pl.BlockSpec((_TSBLK // 128, _FP, 128), lambda i: (i, 0, 0)),
        out_shape=jax.ShapeDtypeStruct((_VP // 128, _FP, 128), jnp.float32),
        scratch_shapes=[pltpu.VMEM((_FP, _TSBLK), jnp.float32)],
    )(wft, embt)


def _mlp_body(x_ref, w1_ref, b1_ref, w2_ref, b2_ref, w3_ref, b3_ref,
              woh_ref, bo_ref, out_ref):
    h = jnp.maximum(jnp.dot(x_ref[...], w1_ref[...],
                            preferred_element_type=jnp.float32) + b1_ref[...], 0.0)
    h = jnp.maximum(jnp.dot(h, w2_ref[...],
                            preferred_element_type=jnp.float32) + b2_ref[...], 0.0)
    h = jnp.maximum(jnp.dot(h, w3_ref[...],
                            preferred_element_type=jnp.float32) + b3_ref[...], 0.0)
    out_ref[...] = jnp.sum(h * woh_ref[...], axis=1) + bo_ref[0, 0]


def _dense_tower(x, W1, b1, W2, b2, W3, b3, woh, bo):
    full = lambda shape: pl.BlockSpec(shape, lambda i: (0,) * len(shape))
    return pl.pallas_call(
        _mlp_body,
        grid=(_B // _BBLK,),
        in_specs=[
            pl.BlockSpec((_BBLK, _DENSE_IN), lambda i: (i, 0)),
            full((_DENSE_IN, 512)), full((1, 512)),
            full((512, 256)), full((1, 256)),
            full((256, _D)), full((1, _D)),
            full((1, _D)), full((1, 1)),
        ],
        out_specs=pl.BlockSpec((_BBLK,), lambda i: (i,)),
        out_shape=jax.ShapeDtypeStruct((_B,), jnp.float32),
    )(x, W1, b1, W2, b2, W3, b3, woh, bo)


def _sc_gather_body(adj_hbm, ts_hbm, hsum_hbm, out_hbm, idx_v, g_v, h_v, o_v, sem):
    w = lax.axis_index("s") * 2 + lax.axis_index("c")
    b0 = w * _BCHUNK
    pltpu.sync_copy(adj_hbm.at[w], idx_v)
    pltpu.sync_copy(hsum_hbm.at[pl.ds(b0, _BCHUNK)], h_v)
    for half in range(2):
        cps = [
            pltpu.async_copy(ts_hbm.at[idx_v.at[half * 13 + f]],
                             g_v.at[half * 13 + f], sem)
            for f in range(13)
        ]
        for cp in cps:
            cp.wait()
    for i in range(_BCHUNK // 16):
        sl = pl.ds(i * 16, 16)
        acc = h_v[sl]
        for f in range(_F):
            acc = acc + g_v[f, sl]
        o_v[sl] = acc
    pltpu.sync_copy(o_v, out_hbm.at[pl.ds(b0, _BCHUNK)])


def _sc_gather():
    return pl.kernel(
        _sc_gather_body,
        out_type=jax.ShapeDtypeStruct((_B,), jnp.float32),
        mesh=plsc.VectorSubcoreMesh(core_axis_name="c", subcore_axis_name="s",
                                    num_cores=2, num_subcores=16),
        scratch_types=[
            pltpu.VMEM((_F, _BCHUNK), jnp.int32),
            pltpu.VMEM((_F, _BCHUNK), jnp.float32),
            pltpu.VMEM((_BCHUNK,), jnp.float32),
            pltpu.VMEM((_BCHUNK,), jnp.float32),
            pltpu.SemaphoreType.DMA,
        ],
    )


def kernel(dense_features, sparse_indices, emb_table, W1, b1, W2, b2, W3, b3,
           Wo, bo):
    wf = Wo[: _F * _D, 0].reshape(_F, _D)
    wft = jnp.zeros((_D, _FP), jnp.float32).at[:, :_F].set(wf.T)
    woh = Wo[_F * _D:, 0].reshape(1, _D)

    ts = _project_table(emb_table.T, wft)
    ts_flat = ts.reshape(_VP * _FP)
    hsum = _dense_tower(dense_features, W1, b1.reshape(1, 512), W2,
                        b2.reshape(1, 256), W3, b3.reshape(1, _D), woh,
                        bo.reshape(1, 1))

    adj = ((sparse_indices & ~jnp.int32(127)) * _FP + (sparse_indices & 127)
           + (jnp.arange(_F, dtype=jnp.int32) << 7)[None, :])
    adj3 = adj.reshape(_NTILES, _BCHUNK, _F).transpose(0, 2, 1)

    out = _sc_gather()(adj3, ts_flat, hsum)
    return out.reshape(_B, 1)


# speedup vs baseline: 1.7056x; 1.1068x over previous
import functools

import jax
import jax.numpy as jnp
from jax import lax
from jax.experimental import pallas as pl
from jax.experimental.pallas import tpu as pltpu
from jax.experimental.pallas import tpu_sc as plsc

_B, _F, _V, _D = 4096, 26, 100000, 32
_DENSE_IN = 13
_FP = 32
_VBLK = 5000
_BBLK = 2048
_NTILES = 32
_BCHUNK = _B // _NTILES


_VP = 102400
_TSBLK = 12800
_TSGRID = _VP // _TSBLK


def _project_body(wft_ref, embt_ref, out_ref, tsb_ref):
    tsb_ref[...] = lax.dot_general(wft_ref[...], embt_ref[...],
                                   dimension_numbers=(((0,), (0,)), ((), ())),
                                   preferred_element_type=jnp.float32)
    for c in range(_TSBLK // 128):
        out_ref[c, :, :] = tsb_ref[:, 128 * c:128 * (c + 1)]


def _project_table(embt, wft):
    return pl.pallas_call(
        _project_body,
        grid=(_TSGRID,),
        in_specs=[
            pl.BlockSpec((_D, _FP), lambda i: (0, 0)),
            pl.BlockSpec((_D, _TSBLK), lambda i: (0, i)),
        ],
        out_specs=pl.BlockSpec((_TSBLK // 128, _FP, 128), lambda i: (i, 0, 0)),
        out_shape=jax.ShapeDtypeStruct((_VP // 128, _FP, 128), jnp.float32),
        scratch_shapes=[pltpu.VMEM((_FP, _TSBLK), jnp.float32)],
    )(wft, embt)


def _mlp_body(x_ref, w1_ref, b1_ref, w2_ref, b2_ref, w3_ref, b3_ref,
              woh_ref, bo_ref, out_ref):
    dot = functools.partial(
        jnp.dot, preferred_element_type=jnp.float32,
        precision=lax.Precision.DEFAULT)
    h = jnp.maximum(dot(x_ref[...], w1_ref[...]) + b1_ref[...], 0.0)
    h = jnp.maximum(dot(h, w2_ref[...]) + b2_ref[...], 0.0)
    h = jnp.maximum(dot(h, w3_ref[...]) + b3_ref[...], 0.0)
    out_ref[...] = jnp.sum(h * woh_ref[...], axis=1) + bo_ref[0, 0]


def _dense_tower(x, W1, b1, W2, b2, W3, b3, woh, bo):
    full = lambda shape: pl.BlockSpec(shape, lambda i: (0,) * len(shape))
    return pl.pallas_call(
        _mlp_body,
        grid=(_B // _BBLK,),
        in_specs=[
            pl.BlockSpec((_BBLK, _DENSE_IN), lambda i: (i, 0)),
            full((_DENSE_IN, 512)), full((1, 512)),
            full((512, 256)), full((1, 256)),
            full((256, _D)), full((1, _D)),
            full((1, _D)), full((1, 1)),
        ],
        out_specs=pl.BlockSpec((_BBLK,), lambda i: (i,)),
        out_shape=jax.ShapeDtypeStruct((_B,), jnp.float32),
    )(x, W1, b1, W2, b2, W3, b3, woh, bo)


def _sc_gather_body(adj_hbm, ts_hbm, hsum_hbm, out_hbm, idx_v, g_v, h_v, o_v, sem):
    w = lax.axis_index("s") * 2 + lax.axis_index("c")
    b0 = w * _BCHUNK
    pltpu.sync_copy(adj_hbm.at[w], idx_v)
    pltpu.sync_copy(hsum_hbm.at[pl.ds(b0, _BCHUNK)], h_v)
    for half in range(2):
        cps = [
            pltpu.async_copy(ts_hbm.at[idx_v.at[half * 13 + f]],
                             g_v.at[half * 13 + f], sem)
            for f in range(13)
        ]
        for cp in cps:
            cp.wait()
    for i in range(_BCHUNK // 16):
        sl = pl.ds(i * 16, 16)
        acc = h_v[sl]
        for f in range(_F):
            acc = acc + g_v[f, sl]
        o_v[sl] = acc
    pltpu.sync_copy(o_v, out_hbm.at[pl.ds(b0, _BCHUNK)])


def _sc_gather():
    return pl.kernel(
        _sc_gather_body,
        out_type=jax.ShapeDtypeStruct((_B,), jnp.float32),
        mesh=plsc.VectorSubcoreMesh(core_axis_name="c", subcore_axis_name="s",
                                    num_cores=2, num_subcores=16),
        scratch_types=[
            pltpu.VMEM((_F, _BCHUNK), jnp.int32),
            pltpu.VMEM((_F, _BCHUNK), jnp.float32),
            pltpu.VMEM((_BCHUNK,), jnp.float32),
            pltpu.VMEM((_BCHUNK,), jnp.float32),
            pltpu.SemaphoreType.DMA,
        ],
    )


def kernel(dense_features, sparse_indices, emb_table, W1, b1, W2, b2, W3, b3,
           Wo, bo):
    wf = Wo[: _F * _D, 0].reshape(_F, _D)
    wft = jnp.zeros((_D, _FP), jnp.float32).at[:, :_F].set(wf.T)
    woh = Wo[_F * _D:, 0].reshape(1, _D)

    ts = _project_table(emb_table.T, wft)
    ts_flat = ts.reshape(_VP * _FP)
    hsum = _dense_tower(dense_features, W1, b1.reshape(1, 512), W2,
                        b2.reshape(1, 256), W3, b3.reshape(1, _D), woh,
                        bo.reshape(1, 1))

    adj = ((sparse_indices & ~jnp.int32(127)) * _FP + (sparse_indices & 127)
           + (jnp.arange(_F, dtype=jnp.int32) << 7)[None, :])
    adj3 = adj.reshape(_NTILES, _BCHUNK, _F).transpose(0, 2, 1)

    out = _sc_gather()(adj3, ts_flat, hsum)
    return out.reshape(_B, 1)


# speedup vs baseline: 1.7777x; 1.0423x over previous
import functools

import jax
import jax.numpy as jnp
from jax import lax
from jax.experimental import pallas as pl
from jax.experimental.pallas import tpu as pltpu
from jax.experimental.pallas import tpu_sc as plsc

_B, _F, _V, _D = 4096, 26, 100000, 32
_DENSE_IN = 13
_FP = 32
_VBLK = 5000
_BBLK = 2048
_NTILES = 32
_BCHUNK = _B // _NTILES


_VP = 102400
_TSBLK = 12800
_TSGRID = _VP // _TSBLK


def _project_body(wft_ref, embt_ref, out_ref, tsb_ref):
    tsb_ref[...] = lax.dot_general(wft_ref[...], embt_ref[...],
                                   dimension_numbers=(((0,), (0,)), ((), ())),
                                   preferred_element_type=jnp.float32)
    for c in range(_TSBLK // 128):
        out_ref[c, :, :] = tsb_ref[:, 128 * c:128 * (c + 1)]


def _project_table(embt, wft):
    return pl.pallas_call(
        _project_body,
        grid=(_TSGRID,),
        in_specs=[
            pl.BlockSpec((_D, _FP), lambda i: (0, 0)),
            pl.BlockSpec((_D, _TSBLK), lambda i: (0, i)),
        ],
        out_specs=pl.BlockSpec((_TSBLK // 128, _FP, 128), lambda i: (i, 0, 0)),
        out_shape=jax.ShapeDtypeStruct((_VP // 128, _FP, 128), jnp.float32),
        scratch_shapes=[pltpu.VMEM((_FP, _TSBLK), jnp.float32)],
    )(wft, embt)


def _mlp_body(x_ref, w1_ref, b1_ref, w2_ref, b2_ref, w3_ref, b3_ref,
              woh_ref, bo_ref, out_ref):
    dot = functools.partial(jnp.dot, preferred_element_type=jnp.float32)
    h = jnp.maximum(dot(x_ref[...], w1_ref[...]) + b1_ref[...], 0.0)
    h = jnp.maximum(dot(h, w2_ref[...]) + b2_ref[...], 0.0)
    h = jnp.maximum(dot(h, w3_ref[...]) + b3_ref[...], 0.0)
    out_ref[...] = jnp.sum(h * woh_ref[...], axis=1) + bo_ref[0, 0]


def _dense_tower(x, W1, b1, W2, b2, W3, b3, woh, bo):
    full = lambda shape: pl.BlockSpec(shape, lambda i: (0,) * len(shape))
    return pl.pallas_call(
        _mlp_body,
        grid=(_B // _BBLK,),
        in_specs=[
            pl.BlockSpec((_BBLK, _DENSE_IN), lambda i: (i, 0)),
            full((_DENSE_IN, 512)), full((1, 512)),
            full((512, 256)), full((1, 256)),
            full((256, _D)), full((1, _D)),
            full((1, _D)), full((1, 1)),
        ],
        out_specs=pl.BlockSpec((_BBLK,), lambda i: (i,)),
        out_shape=jax.ShapeDtypeStruct((_B,), jnp.float32),
    )(x, W1, b1, W2, b2, W3, b3, woh, bo)


def _sc_gather_body(idxt_hbm, ts_hbm, hsum_hbm, out_hbm, raw_v, idx_v, g_v,
                    h_v, o_v, sem):
    w = lax.axis_index("s") * 2 + lax.axis_index("c")
    b0 = w * _BCHUNK
    pltpu.sync_copy(idxt_hbm.at[:, pl.ds(b0, _BCHUNK)], raw_v)
    pltpu.sync_copy(hsum_hbm.at[pl.ds(b0, _BCHUNK)], h_v)
    cps = []
    for f in range(_F):
        for i in range(_BCHUNK // 16):
            sl = pl.ds(i * 16, 16)
            v = raw_v[f, sl]
            idx_v[f, sl] = ((v >> 7) << 12) + (v & 127) + (f << 7)
        cps.append(pltpu.async_copy(ts_hbm.at[idx_v.at[f]], g_v.at[f], sem))
    for cp in cps:
        cp.wait()
    for i in range(_BCHUNK // 16):
        sl = pl.ds(i * 16, 16)
        acc = h_v[sl]
        for f in range(_F):
            acc = acc + g_v[f, sl]
        o_v[sl] = acc
    pltpu.sync_copy(o_v, out_hbm.at[pl.ds(b0, _BCHUNK)])


def _sc_gather():
    return pl.kernel(
        _sc_gather_body,
        out_type=jax.ShapeDtypeStruct((_B,), jnp.float32),
        mesh=plsc.VectorSubcoreMesh(core_axis_name="c", subcore_axis_name="s",
                                    num_cores=2, num_subcores=16),
        scratch_types=[
            pltpu.VMEM((_F, _BCHUNK), jnp.int32),
            pltpu.VMEM((_F, _BCHUNK), jnp.int32),
            pltpu.VMEM((_F, _BCHUNK), jnp.float32),
            pltpu.VMEM((_BCHUNK,), jnp.float32),
            pltpu.VMEM((_BCHUNK,), jnp.float32),
            pltpu.SemaphoreType.DMA,
        ],
    )


def kernel(dense_features, sparse_indices, emb_table, W1, b1, W2, b2, W3, b3,
           Wo, bo):
    wf = Wo[: _F * _D, 0].reshape(_F, _D)
    wft = jnp.zeros((_D, _FP), jnp.float32).at[:, :_F].set(wf.T)
    woh = Wo[_F * _D:, 0].reshape(1, _D)

    ts = _project_table(emb_table.T, wft)
    ts_flat = ts.reshape(_VP * _FP)
    hsum = _dense_tower(dense_features, W1, b1.reshape(1, 512), W2,
                        b2.reshape(1, 256), W3, b3.reshape(1, _D), woh,
                        bo.reshape(1, 1))

    out = _sc_gather()(sparse_indices.T, ts_flat, hsum)
    return out.reshape(_B, 1)


# speedup vs baseline: 2.1493x; 1.2090x over previous
import functools

import jax
import jax.numpy as jnp
from jax import lax
from jax.experimental import pallas as pl
from jax.experimental.pallas import tpu as pltpu
from jax.experimental.pallas import tpu_sc as plsc

_B, _F, _V, _D = 4096, 26, 100000, 32
_DENSE_IN = 13
_FP = 32
_VBLK = 5000
_BBLK = 2048
_NTILES = 32
_BCHUNK = _B // _NTILES


_VP = 102400
_TSBLK = 12800
_TSGRID = _VP // _TSBLK


def _project_body(wft_ref, embt_ref, out_ref, tsb_ref):
    tsb_ref[...] = lax.dot_general(wft_ref[...], embt_ref[...],
                                   dimension_numbers=(((0,), (0,)), ((), ())),
                                   preferred_element_type=jnp.float32)
    for c in range(_TSBLK // 128):
        out_ref[c, :, :] = tsb_ref[:, 128 * c:128 * (c + 1)]


def _project_table(embt, wft):
    return pl.pallas_call(
        _project_body,
        grid=(_TSGRID,),
        in_specs=[
            pl.BlockSpec((_D, _FP), lambda i: (0, 0)),
            pl.BlockSpec((_D, _TSBLK), lambda i: (0, i)),
        ],
        out_specs=pl.BlockSpec((_TSBLK // 128, _FP, 128), lambda i: (i, 0, 0)),
        out_shape=jax.ShapeDtypeStruct((_VP // 128, _FP, 128), jnp.float32),
        scratch_shapes=[pltpu.VMEM((_FP, _TSBLK), jnp.float32)],
    )(wft, embt)


def _mlp_body(xt_ref, w1_ref, b1_ref, w2_ref, b2_ref, w3_ref, b3_ref,
              woh_ref, bo_ref, sp_ref, out_ref):
    dg = lambda a, b: lax.dot_general(a, b, (((0,), (0,)), ((), ())),
                                      preferred_element_type=jnp.float32)
    bf = jnp.bfloat16
    h = jnp.maximum(dg(w1_ref[...], xt_ref[...]) + b1_ref[...], 0.0)
    w2 = w2_ref[...]
    w2h = w2.astype(bf)
    w2l = (w2 - w2h.astype(jnp.float32)).astype(bf)
    hh = h.astype(bf)
    hl = (h - hh.astype(jnp.float32)).astype(bf)
    h2 = dg(w2h, hh) + (dg(w2h, hl) + dg(w2l, hh))
    h = jnp.maximum(h2 + b2_ref[...], 0.0)
    h = jnp.maximum(dg(w3_ref[...], h) + b3_ref[...], 0.0)
    out_ref[...] = (jnp.sum(h * woh_ref[...], axis=0) + bo_ref[0, 0]
                    + sp_ref[...])


def _dense_tower(xt, W1, b1, W2, b2, W3, b3, woh, bo, sp):
    full = lambda shape: pl.BlockSpec(shape, lambda i: (0,) * len(shape))
    return pl.pallas_call(
        _mlp_body,
        grid=(_B // _BBLK,),
        in_specs=[
            pl.BlockSpec((_DENSE_IN, _BBLK), lambda i: (0, i)),
            full((_DENSE_IN, 512)), full((512, 1)),
            full((512, 256)), full((256, 1)),
            full((256, _D)), full((_D, 1)),
            full((_D, 1)), full((1, 1)),
            pl.BlockSpec((_BBLK,), lambda i: (i,)),
        ],
        out_specs=pl.BlockSpec((_BBLK,), lambda i: (i,)),
        out_shape=jax.ShapeDtypeStruct((_B,), jnp.float32),
    )(xt, W1, b1, W2, b2, W3, b3, woh, bo, sp)


def _sc_gather_body(idxt_hbm, ts_hbm, out_hbm, raw_v, idx_v, g_v, o_v, sem):
    w = lax.axis_index("s") * 2 + lax.axis_index("c")
    b0 = w * _BCHUNK
    pltpu.sync_copy(idxt_hbm.at[:, pl.ds(b0, _BCHUNK)], raw_v)
    cps = []
    for f in range(_F):
        for i in range(_BCHUNK // 16):
            sl = pl.ds(i * 16, 16)
            v = raw_v[f, sl]
            idx_v[f, sl] = ((v >> 7) << 12) + (v & 127) + (f << 7)
        cps.append(pltpu.async_copy(ts_hbm.at[idx_v.at[f]], g_v.at[f], sem))
    for cp in cps:
        cp.wait()
    for i in range(_BCHUNK // 16):
        sl = pl.ds(i * 16, 16)
        acc = g_v[0, sl]
        for f in range(1, _F):
            acc = acc + g_v[f, sl]
        o_v[sl] = acc
    pltpu.sync_copy(o_v, out_hbm.at[pl.ds(b0, _BCHUNK)])


def _sc_gather():
    return pl.kernel(
        _sc_gather_body,
        out_type=jax.ShapeDtypeStruct((_B,), jnp.float32),
        mesh=plsc.VectorSubcoreMesh(core_axis_name="c", subcore_axis_name="s",
                                    num_cores=2, num_subcores=16),
        scratch_types=[
            pltpu.VMEM((_F, _BCHUNK), jnp.int32),
            pltpu.VMEM((_F, _BCHUNK), jnp.int32),
            pltpu.VMEM((_F, _BCHUNK), jnp.float32),
            pltpu.VMEM((_BCHUNK,), jnp.float32),
            pltpu.SemaphoreType.DMA,
        ],
    )


def kernel(dense_features, sparse_indices, emb_table, W1, b1, W2, b2, W3, b3,
           Wo, bo):
    wf = Wo[: _F * _D, 0].reshape(_F, _D)
    wft = jnp.zeros((_D, _FP), jnp.float32).at[:, :_F].set(wf.T)
    woh = Wo[_F * _D:, 0].reshape(_D, 1)

    ts = _project_table(emb_table.T, wft)
    ts_flat = ts.reshape(_VP * _FP)

    sp = _sc_gather()(sparse_indices.T, ts_flat)

    out = _dense_tower(dense_features.T, W1, b1.reshape(512, 1), W2,
                       b2.reshape(256, 1), W3, b3.reshape(_D, 1), woh,
                       bo.reshape(1, 1), sp)
    return out.reshape(_B, 1)


# speedup vs baseline: 2.2125x; 1.0294x over previous
import functools

import jax
import jax.numpy as jnp
from jax import lax
from jax.experimental import pallas as pl
from jax.experimental.pallas import tpu as pltpu
from jax.experimental.pallas import tpu_sc as plsc

_B, _F, _V, _D = 4096, 26, 100000, 32
_DENSE_IN = 13
_FP = 32
_VBLK = 5000
_BBLK = 2048
_NTILES = 32
_BCHUNK = _B // _NTILES


_VP = 102400
_TSBLK = 12800
_TSGRID = _VP // _TSBLK


def _project_body(wft_ref, embt_ref, out_ref, tsb_ref):
    tsb_ref[...] = lax.dot_general(wft_ref[...], embt_ref[...],
                                   dimension_numbers=(((0,), (0,)), ((), ())),
                                   preferred_element_type=jnp.float32)
    for c in range(_TSBLK // 128):
        out_ref[c, :, :] = tsb_ref[:, 128 * c:128 * (c + 1)]


def _project_table(embt, wft):
    return pl.pallas_call(
        _project_body,
        grid=(_TSGRID,),
        in_specs=[
            pl.BlockSpec((_D, _FP), lambda i: (0, 0)),
            pl.BlockSpec((_D, _TSBLK), lambda i: (0, i)),
        ],
        out_specs=pl.BlockSpec((_TSBLK // 128, _FP, 128), lambda i: (i, 0, 0)),
        out_shape=jax.ShapeDtypeStruct((_VP // 128, _FP, 128), jnp.float32),
        scratch_shapes=[pltpu.VMEM((_FP, _TSBLK), jnp.float32)],
    )(wft, embt)


def _mlp_body(xt_ref, w1_ref, b1_ref, w2_ref, b2_ref, w3_ref, b3_ref,
              woh_ref, bo_ref, out_ref):
    dg = lambda a, b: lax.dot_general(a, b, (((0,), (0,)), ((), ())),
                                      preferred_element_type=jnp.float32)
    bf = jnp.bfloat16
    h = jnp.maximum(dg(w1_ref[...], xt_ref[...]) + b1_ref[...], 0.0)
    w2 = w2_ref[...]
    w2h = w2.astype(bf)
    w2l = (w2 - w2h.astype(jnp.float32)).astype(bf)
    hh = h.astype(bf)
    hl = (h - hh.astype(jnp.float32)).astype(bf)
    h2 = dg(w2h, hh) + (dg(w2h, hl) + dg(w2l, hh))
    h = jnp.maximum(h2 + b2_ref[...], 0.0)
    h = jnp.maximum(dg(w3_ref[...], h) + b3_ref[...], 0.0)
    out_ref[...] = jnp.sum(h * woh_ref[...], axis=0) + bo_ref[0, 0]


def _dense_tower(xt, W1, b1, W2, b2, W3, b3, woh, bo):
    full = lambda shape: pl.BlockSpec(shape, lambda i: (0,) * len(shape))
    return pl.pallas_call(
        _mlp_body,
        grid=(_B // _BBLK,),
        in_specs=[
            pl.BlockSpec((_DENSE_IN, _BBLK), lambda i: (0, i)),
            full((_DENSE_IN, 512)), full((512, 1)),
            full((512, 256)), full((256, 1)),
            full((256, _D)), full((_D, 1)),
            full((_D, 1)), full((1, 1)),
        ],
        out_specs=pl.BlockSpec((_BBLK,), lambda i: (i,)),
        out_shape=jax.ShapeDtypeStruct((_B,), jnp.float32),
    )(xt, W1, b1, W2, b2, W3, b3, woh, bo)


def _final_add_body(a_ref, b_ref, out_ref):
    out_ref[...] = a_ref[...] + b_ref[...]


def _final_add(a, b):
    return pl.pallas_call(
        _final_add_body,
        in_specs=[pl.BlockSpec((_B,), lambda: (0,)),
                  pl.BlockSpec((_B,), lambda: (0,))],
        out_specs=pl.BlockSpec((_B,), lambda: (0,)),
        out_shape=jax.ShapeDtypeStruct((_B,), jnp.float32),
    )(a, b)


def _sc_gather_body(idxt_hbm, ts_hbm, out_hbm, raw_v, idx_v, g_v, o_v, sem):
    w = lax.axis_index("s") * 2 + lax.axis_index("c")
    b0 = w * _BCHUNK
    pltpu.sync_copy(idxt_hbm.at[:, pl.ds(b0, _BCHUNK)], raw_v)
    cps = []
    for f in range(_F):
        for i in range(_BCHUNK // 16):
            sl = pl.ds(i * 16, 16)
            v = raw_v[f, sl]
            idx_v[f, sl] = ((v >> 7) << 12) + (v & 127) + (f << 7)
        cps.append(pltpu.async_copy(ts_hbm.at[idx_v.at[f]], g_v.at[f], sem))
    for cp in cps:
        cp.wait()
    for i in range(_BCHUNK // 16):
        sl = pl.ds(i * 16, 16)
        acc = g_v[0, sl]
        for f in range(1, _F):
            acc = acc + g_v[f, sl]
        o_v[sl] = acc
    pltpu.sync_copy(o_v, out_hbm.at[pl.ds(b0, _BCHUNK)])


def _sc_gather():
    return pl.kernel(
        _sc_gather_body,
        out_type=jax.ShapeDtypeStruct((_B,), jnp.float32),
        mesh=plsc.VectorSubcoreMesh(core_axis_name="c", subcore_axis_name="s",
                                    num_cores=2, num_subcores=16),
        scratch_types=[
            pltpu.VMEM((_F, _BCHUNK), jnp.int32),
            pltpu.VMEM((_F, _BCHUNK), jnp.int32),
            pltpu.VMEM((_F, _BCHUNK), jnp.float32),
            pltpu.VMEM((_BCHUNK,), jnp.float32),
            pltpu.SemaphoreType.DMA,
        ],
    )


def kernel(dense_features, sparse_indices, emb_table, W1, b1, W2, b2, W3, b3,
           Wo, bo):
    wf = Wo[: _F * _D, 0].reshape(_F, _D)
    wft = jnp.zeros((_D, _FP), jnp.float32).at[:, :_F].set(wf.T)
    woh = Wo[_F * _D:, 0].reshape(_D, 1)

    ts = _project_table(emb_table.T, wft)
    ts_flat = ts.reshape(_VP * _FP)

    sp = _sc_gather()(sparse_indices.T, ts_flat)

    hsum = _dense_tower(dense_features.T, W1, b1.reshape(512, 1), W2,
                        b2.reshape(256, 1), W3, b3.reshape(_D, 1), woh,
                        bo.reshape(1, 1))

    return _final_add(sp, hsum).reshape(_B, 1)


# speedup vs baseline: 2.3629x; 1.0680x over previous
import functools

import jax
import jax.numpy as jnp
from jax import lax
from jax.experimental import pallas as pl
from jax.experimental.pallas import tpu as pltpu
from jax.experimental.pallas import tpu_sc as plsc

_B, _F, _V, _D = 4096, 26, 100000, 32
_DENSE_IN = 13
_FP = 32
_VBLK = 5000
_BBLK = 2048
_NTILES = 32
_BCHUNK = _B // _NTILES


_VP = 102400
_TSBLK = 12800
_TSGRID = _VP // _TSBLK


def _project_body(wft_ref, embt_ref, out_ref, tsb_ref):
    tsb_ref[...] = lax.dot_general(wft_ref[...], embt_ref[...],
                                   dimension_numbers=(((0,), (0,)), ((), ())),
                                   preferred_element_type=jnp.float32)
    for c in range(_TSBLK // 128):
        out_ref[c, :, :] = tsb_ref[:, 128 * c:128 * (c + 1)]


def _project_table(embt, wft):
    return pl.pallas_call(
        _project_body,
        grid=(_TSGRID,),
        in_specs=[
            pl.BlockSpec((_D, _FP), lambda i: (0, 0)),
            pl.BlockSpec((_D, _TSBLK), lambda i: (0, i)),
        ],
        out_specs=pl.BlockSpec((_TSBLK // 128, _FP, 128), lambda i: (i, 0, 0)),
        out_shape=jax.ShapeDtypeStruct((_VP // 128, _FP, 128), jnp.float32),
        scratch_shapes=[pltpu.VMEM((_FP, _TSBLK), jnp.float32)],
    )(wft, embt)


def _mlp_body(xt_ref, w1_ref, b1_ref, w2_ref, b2_ref, w3_ref, b3_ref,
              woh_ref, bo_ref, out_ref):
    dg = lambda a, b: lax.dot_general(a, b, (((0,), (0,)), ((), ())),
                                      preferred_element_type=jnp.float32)
    bf = jnp.bfloat16
    h = jnp.maximum(dg(w1_ref[...], xt_ref[...]) + b1_ref[...], 0.0)
    h2 = dg(w2_ref[...], h)
    h = jnp.maximum(h2 + b2_ref[...], 0.0)
    h = jnp.maximum(dg(w3_ref[...], h) + b3_ref[...], 0.0)
    out_ref[...] = jnp.sum(h * woh_ref[...], axis=0) + bo_ref[0, 0]


def _dense_tower(xt, W1, b1, W2, b2, W3, b3, woh, bo):
    full = lambda shape: pl.BlockSpec(shape, lambda i: (0,) * len(shape))
    return pl.pallas_call(
        _mlp_body,
        grid=(_B // _BBLK,),
        in_specs=[
            pl.BlockSpec((_DENSE_IN, _BBLK), lambda i: (0, i)),
            full((_DENSE_IN, 512)), full((512, 1)),
            full((512, 256)), full((256, 1)),
            full((256, _D)), full((_D, 1)),
            full((_D, 1)), full((1, 1)),
        ],
        out_specs=pl.BlockSpec((_BBLK,), lambda i: (i,)),
        out_shape=jax.ShapeDtypeStruct((_B,), jnp.float32),
    )(xt, W1, b1, W2, b2, W3, b3, woh, bo)


def _final_add_body(a_ref, b_ref, out_ref):
    out_ref[...] = a_ref[...] + b_ref[...]


def _final_add(a, b):
    return pl.pallas_call(
        _final_add_body,
        in_specs=[pl.BlockSpec((_B,), lambda: (0,)),
                  pl.BlockSpec((_B,), lambda: (0,))],
        out_specs=pl.BlockSpec((_B,), lambda: (0,)),
        out_shape=jax.ShapeDtypeStruct((_B,), jnp.float32),
    )(a, b)


def _sc_gather_body(idxt_hbm, ts_hbm, out_hbm, raw_v, idx_v, g_v, o_v, sem):
    w = lax.axis_index("s") * 2 + lax.axis_index("c")
    b0 = w * _BCHUNK
    pltpu.sync_copy(idxt_hbm.at[:, pl.ds(b0, _BCHUNK)], raw_v)
    cps = []
    for f in range(_F):
        for i in range(_BCHUNK // 16):
            sl = pl.ds(i * 16, 16)
            v = raw_v[f, sl]
            idx_v[f, sl] = ((v >> 7) << 12) + (v & 127) + (f << 7)
        cps.append(pltpu.async_copy(ts_hbm.at[idx_v.at[f]], g_v.at[f], sem))
    for cp in cps:
        cp.wait()
    for i in range(_BCHUNK // 16):
        sl = pl.ds(i * 16, 16)
        acc = g_v[0, sl]
        for f in range(1, _F):
            acc = acc + g_v[f, sl]
        o_v[sl] = acc
    pltpu.sync_copy(o_v, out_hbm.at[pl.ds(b0, _BCHUNK)])


def _sc_gather():
    return pl.kernel(
        _sc_gather_body,
        out_type=jax.ShapeDtypeStruct((_B,), jnp.float32),
        mesh=plsc.VectorSubcoreMesh(core_axis_name="c", subcore_axis_name="s",
                                    num_cores=2, num_subcores=16),
        scratch_types=[
            pltpu.VMEM((_F, _BCHUNK), jnp.int32),
            pltpu.VMEM((_F, _BCHUNK), jnp.int32),
            pltpu.VMEM((_F, _BCHUNK), jnp.float32),
            pltpu.VMEM((_BCHUNK,), jnp.float32),
            pltpu.SemaphoreType.DMA,
        ],
    )


def kernel(dense_features, sparse_indices, emb_table, W1, b1, W2, b2, W3, b3,
           Wo, bo):
    wf = Wo[: _F * _D, 0].reshape(_F, _D)
    wft = jnp.zeros((_D, _FP), jnp.float32).at[:, :_F].set(wf.T)
    woh = Wo[_F * _D:, 0].reshape(_D, 1)

    ts = _project_table(emb_table.T, wft)
    ts_flat = ts.reshape(_VP * _FP)

    sp = _sc_gather()(sparse_indices.T, ts_flat)

    hsum = _dense_tower(dense_features.T, W1, b1.reshape(512, 1), W2,
                        b2.reshape(256, 1), W3, b3.reshape(_D, 1), woh,
                        bo.reshape(1, 1))

    return _final_add(sp, hsum).reshape(_B, 1)


# speedup vs baseline: 2.3639x; 1.0004x over previous
import functools

import jax
import jax.numpy as jnp
from jax import lax
from jax.experimental import pallas as pl
from jax.experimental.pallas import tpu as pltpu
from jax.experimental.pallas import tpu_sc as plsc

_B, _F, _V, _D = 4096, 26, 100000, 32
_DENSE_IN = 13
_FP = 32
_VBLK = 5000
_BBLK = 2048
_NTILES = 32
_BCHUNK = _B // _NTILES


_VP = 102400
_TSBLK = 12800
_TSGRID = _VP // _TSBLK


def _project_body(wft_ref, embt_ref, out_ref, tsb_ref):
    tsb_ref[...] = lax.dot_general(wft_ref[...], embt_ref[...],
                                   dimension_numbers=(((0,), (0,)), ((), ())),
                                   preferred_element_type=jnp.float32)
    for c in range(_TSBLK // 128):
        out_ref[c, :, :] = tsb_ref[:, 128 * c:128 * (c + 1)]


def _project_table(embt, wft):
    return pl.pallas_call(
        _project_body,
        grid=(_TSGRID,),
        in_specs=[
            pl.BlockSpec((_D, _FP), lambda i: (0, 0)),
            pl.BlockSpec((_D, _TSBLK), lambda i: (0, i)),
        ],
        out_specs=pl.BlockSpec((_TSBLK // 128, _FP, 128), lambda i: (i, 0, 0)),
        out_shape=jax.ShapeDtypeStruct((_VP // 128, _FP, 128), jnp.float32),
        scratch_shapes=[pltpu.VMEM((_FP, _TSBLK), jnp.float32)],
    )(wft, embt)


def _mlp_body(xt_ref, w1_ref, b1_ref, w2_ref, b2_ref, w3_ref, b3_ref,
              woh_ref, bo_ref, out_ref):
    dg = lambda a, b: lax.dot_general(a, b, (((0,), (0,)), ((), ())),
                                      preferred_element_type=jnp.float32)
    h = jnp.maximum(dg(w1_ref[...], xt_ref[...]) + b1_ref[...], 0.0)
    h = jnp.maximum(dg(w2_ref[...], h) + b2_ref[...], 0.0)
    h = jnp.maximum(dg(w3_ref[...], h) + b3_ref[...], 0.0)
    out_ref[...] = jnp.sum(h * woh_ref[...], axis=0) + bo_ref[0, 0]


def _dense_tower(xt, W1, b1, W2, b2, W3, b3, woh, bo):
    full = lambda shape: pl.BlockSpec(shape, lambda i: (0,) * len(shape))
    return pl.pallas_call(
        _mlp_body,
        grid=(_B // _BBLK,),
        in_specs=[
            pl.BlockSpec((_DENSE_IN, _BBLK), lambda i: (0, i)),
            full((_DENSE_IN, 512)), full((512, 1)),
            full((512, 256)), full((256, 1)),
            full((256, _D)), full((_D, 1)),
            full((_D, 1)), full((1, 1)),
        ],
        out_specs=pl.BlockSpec((_BBLK,), lambda i: (i,)),
        out_shape=jax.ShapeDtypeStruct((_B,), jnp.float32),
    )(xt, W1, b1, W2, b2, W3, b3, woh, bo)


def _final_add_body(a_ref, b_ref, out_ref):
    out_ref[...] = a_ref[...] + b_ref[...]


def _final_add(a, b):
    return pl.pallas_call(
        _final_add_body,
        in_specs=[pl.BlockSpec((_B,), lambda: (0,)),
                  pl.BlockSpec((_B,), lambda: (0,))],
        out_specs=pl.BlockSpec((_B,), lambda: (0,)),
        out_shape=jax.ShapeDtypeStruct((_B,), jnp.float32),
    )(a, b)


def _sc_gather_body(idxt_hbm, ts_hbm, out_hbm, raw_v, idx_v, g_v, o_v, sem):
    w = lax.axis_index("s") * 2 + lax.axis_index("c")
    b0 = w * _BCHUNK
    pltpu.sync_copy(idxt_hbm.at[:, pl.ds(b0, _BCHUNK)], raw_v)
    cps = []
    for f in range(_F):
        for i in range(_BCHUNK // 16):
            sl = pl.ds(i * 16, 16)
            v = raw_v[f, sl]
            idx_v[f, sl] = ((v >> 7) << 12) + (v & 127) + (f << 7)
        cps.append(pltpu.async_copy(ts_hbm.at[idx_v.at[f]], g_v.at[f], sem))
    for cp in cps:
        cp.wait()
    for i in range(_BCHUNK // 16):
        sl = pl.ds(i * 16, 16)
        acc = g_v[0, sl]
        for f in range(1, _F):
            acc = acc + g_v[f, sl]
        o_v[sl] = acc
    pltpu.sync_copy(o_v, out_hbm.at[pl.ds(b0, _BCHUNK)])


def _sc_gather():
    return pl.kernel(
        _sc_gather_body,
        out_type=jax.ShapeDtypeStruct((_B,), jnp.float32),
        mesh=plsc.VectorSubcoreMesh(core_axis_name="c", subcore_axis_name="s",
                                    num_cores=2, num_subcores=16),
        scratch_types=[
            pltpu.VMEM((_F, _BCHUNK), jnp.int32),
            pltpu.VMEM((_F, _BCHUNK), jnp.int32),
            pltpu.VMEM((_F, _BCHUNK), jnp.float32),
            pltpu.VMEM((_BCHUNK,), jnp.float32),
            pltpu.SemaphoreType.DMA,
        ],
    )


def kernel(dense_features, sparse_indices, emb_table, W1, b1, W2, b2, W3, b3,
           Wo, bo):
    wf = Wo[: _F * _D, 0].reshape(_F, _D)
    wft = jnp.zeros((_D, _FP), jnp.float32).at[:, :_F].set(wf.T)
    woh = Wo[_F * _D:, 0].reshape(_D, 1)

    ts = _project_table(emb_table.T, wft)
    ts_flat = ts.reshape(_VP * _FP)

    sp = _sc_gather()(sparse_indices.T, ts_flat)

    hsum = _dense_tower(dense_features.T, W1, b1.reshape(512, 1), W2,
                        b2.reshape(256, 1), W3, b3.reshape(_D, 1), woh,
                        bo.reshape(1, 1))

    return _final_add(sp, hsum).reshape(_B, 1)
